# baseline (speedup 1.0000x reference)
import functools

import jax
import jax.numpy as jnp
from jax import lax
from jax.experimental import pallas as pl
from jax.experimental.pallas import tpu as pltpu
from jax.experimental.pallas import tpu_sc as plsc
from jax._src.pallas import mpmd as _pl_mpmd

M = 1_000_000
D = 32
B = 16384
LR = 0.01

NC = 2
NS = 16
BT = B // NS
ACC = 43008
R = 43000
TRASH = R
P = 12
ZR = ACC // NS


def _rsqrt16(x):
    i = plsc.bitcast(x, jnp.int32)
    i = jnp.int32(0x5F3759DF) - lax.shift_right_logical(i, 1)
    y = plsc.bitcast(i, jnp.float32)
    for _ in range(3):
        y = y * (jnp.float32(1.5) - jnp.float32(0.5) * x * y * y)
    return y


@functools.cache
def _build_sc_update():
    mesh = plsc.VectorSubcoreMesh(core_axis_name="c", subcore_axis_name="s")

    def _sc_update(items, idx2, val, out, stage, acc, sem_a, sem_b):
        @pl.with_scoped(
            pltpu.VMEM((8, 128), jnp.int32),
            pltpu.VMEM((BT,), jnp.int32),
            pltpu.VMEM((BT,), jnp.int32),
            pltpu.VMEM((BT,), jnp.int32),
            pltpu.VMEM((BT, D), jnp.float32),
            pltpu.VMEM((32, D), jnp.float32),
            pltpu.VMEM((128, D), jnp.float32),
        )
        def _body(idx_v, gidx, aidx, tidx, buf, vbuf, zrow):
            _tile_body(idx2, val, out, stage, acc, sem_a, sem_b,
                       idx_v, gidx, aidx, tidx, buf, vbuf, zrow)

        _body()

    def _tile_body(idx2, val, out, stage, acc, sem_a, sem_b,
                   idx_v, gidx, aidx, tidx, buf, vbuf, zrow):
        c = lax.axis_index("c")
        s = lax.axis_index("s")

        pltpu.sync_copy(idx2.at[pl.ds(s * 8, 8)], idx_v)

        zero16 = jnp.zeros((16,), jnp.float32)

        def zrow_body(i, carry):
            zrow[i, pl.ds(0, 16)] = zero16
            zrow[i, pl.ds(16, 16)] = zero16
            return carry

        lax.fori_loop(0, 128, zrow_body, 0)

        srow = c * B + s * BT

        def clip_row(r, carry):
            v0 = vbuf[r, pl.ds(0, 16)]
            v1 = vbuf[r, pl.ds(16, 16)]
            ssum = jnp.sum(v0 * v0 + v1 * v1)
            ssum16 = jnp.broadcast_to(ssum, (16,))
            scale = jnp.float32(-LR) * jnp.minimum(
                jnp.float32(1.0),
                _rsqrt16(jnp.maximum(ssum16, jnp.float32(1e-24))))
            vbuf[r, pl.ds(0, 16)] = v0 * scale
            vbuf[r, pl.ds(16, 16)] = v1 * scale
            return carry

        def neg_row(r, carry):
            vbuf[r, pl.ds(0, 16)] = -vbuf[r, pl.ds(0, 16)]
            vbuf[r, pl.ds(16, 16)] = -vbuf[r, pl.ds(16, 16)]
            return carry

        def clip_chunk(k, carry):
            pltpu.sync_copy(val.at[pl.ds(s * BT + k * 32, 32)], vbuf)
            lax.fori_loop(0, 32, clip_row, 0)
            pltpu.sync_copy(vbuf, stage.at[pl.ds(srow + k * 32, 32)])
            lax.fori_loop(0, 32, neg_row, 0)
            pltpu.sync_copy(
                vbuf, stage.at[pl.ds(NC * B + srow + k * 32, 32)])
            return carry

        lax.fori_loop(0, BT // 32, clip_chunk, 0)

        def accz_body(i, carry):
            pltpu.sync_copy(zrow, acc.at[pl.ds(s * ZR + i * 128, 128)])
            return carry

        lax.fori_loop(0, ZR // 128, accz_body, 0)

        plsc.subcore_barrier()

        def pass_body(p, carry):
            lo = (c * P + p) * R

            def map_k(k, carry2):
                def map_o(o, carry3):
                    iv = idx_v[k, pl.ds(o * 16, 16)]
                    inr = (iv >= lo) & (iv < lo + R)
                    loc = iv - lo
                    g = k * 128 + o * 16
                    gidx[pl.ds(g, 16)] = jnp.where(inr, iv, lo)
                    aidx[pl.ds(g, 16)] = jnp.where(inr, loc, 0)
                    tidx[pl.ds(g, 16)] = jnp.where(inr, loc, TRASH)
                    return carry3
                return lax.fori_loop(0, 8, map_o, carry2)

            lax.fori_loop(0, 8, map_k, 0)

            pltpu.sync_copy(stage.at[pl.ds(srow, BT)], buf)
            pltpu.async_copy(buf, acc.at[tidx], sem_a, add=True).wait()
            plsc.subcore_barrier()

            pltpu.async_copy(out.at[gidx], buf, sem_a).wait()
            pltpu.async_copy(acc.at[aidx], buf, sem_b, add=True).wait()
            plsc.subcore_barrier()

            pltpu.async_copy(buf, out.at[gidx], sem_b).wait()

            pltpu.sync_copy(stage.at[pl.ds(NC * B + srow, BT)], buf)
            pltpu.async_copy(buf, acc.at[tidx], sem_a, add=True).wait()
            plsc.subcore_barrier()
            return carry

        lax.fori_loop(0, P, pass_body, 0)

    return _pl_mpmd._mpmd_map(
        [(mesh, _sc_update)],
        out_types=(
            jax.ShapeDtypeStruct((M, D), jnp.float32),
            jax.ShapeDtypeStruct((2 * NC * B, D), jnp.float32),
        ),
        input_output_aliases={0: 0},
        scratch_types=[
            pltpu.VMEM_SHARED((ACC, D), jnp.float32),
            pltpu.SemaphoreType.DMA,
            pltpu.SemaphoreType.DMA,
        ],
        compiler_params=pltpu.CompilerParams(
            needs_layout_passes=False, use_tc_tiling_on_sc=False),
    )


def kernel(items_emb, idx, val):
    idx2 = idx.astype(jnp.int32).reshape(128, 128)
    out, _ = _build_sc_update()(items_emb, idx2, val)
    return out


# speedup vs baseline: 2.1994x; 2.1994x over previous
import functools

import jax
import jax.numpy as jnp
from jax import lax
from jax.experimental import pallas as pl
from jax.experimental.pallas import tpu as pltpu
from jax.experimental.pallas import tpu_sc as plsc
from jax._src.pallas import mpmd as _pl_mpmd

M = 1_000_000
D = 32
B = 16384
LR = 0.01

NC = 2
NS = 16
BT = B // NS
ACC = 43008
W = 1024
R = 41984
P = 12
ZR = ACC // NS


def _rsqrt16(x):
    i = plsc.bitcast(x, jnp.int32)
    i = jnp.int32(0x5F3759DF) - lax.shift_right_logical(i, 1)
    y = plsc.bitcast(i, jnp.float32)
    for _ in range(3):
        y = y * (jnp.float32(1.5) - jnp.float32(0.5) * x * y * y)
    return y


@functools.cache
def _build_sc_update():
    mesh = plsc.VectorSubcoreMesh(core_axis_name="c", subcore_axis_name="s")

    def _sc_update(items, pristine, idx2, val, out, stage, acc, sem_a, sem_b):
        @pl.with_scoped(
            pltpu.VMEM((8, 128), jnp.int32),
            pltpu.VMEM((BT,), jnp.int32),
            pltpu.VMEM((BT,), jnp.int32),
            pltpu.VMEM((BT,), jnp.int32),
            pltpu.VMEM((BT, D), jnp.float32),
            pltpu.VMEM((32, D), jnp.float32),
            pltpu.VMEM((128, D), jnp.float32),
        )
        def _body(idx_v, gidx, aidx, tidx, buf, vbuf, zrow):
            _tile_body(pristine, idx2, val, out, stage, acc, sem_a, sem_b,
                       idx_v, gidx, aidx, tidx, buf, vbuf, zrow)

        _body()

    def _tile_body(pristine, idx2, val, out, stage, acc, sem_a, sem_b,
                   idx_v, gidx, aidx, tidx, buf, vbuf, zrow):
        c = lax.axis_index("c")
        s = lax.axis_index("s")

        pltpu.sync_copy(idx2.at[pl.ds(s * 8, 8)], idx_v)

        zero16 = jnp.zeros((16,), jnp.float32)

        def zrow_body(i, carry):
            zrow[i, pl.ds(0, 16)] = zero16
            zrow[i, pl.ds(16, 16)] = zero16
            return carry

        lax.fori_loop(0, 128, zrow_body, 0)

        srow = c * B + s * BT

        def clip_row(r, carry):
            v0 = vbuf[r, pl.ds(0, 16)]
            v1 = vbuf[r, pl.ds(16, 16)]
            ssum = jnp.sum(v0 * v0 + v1 * v1)
            ssum16 = jnp.broadcast_to(ssum, (16,))
            scale = jnp.float32(-LR) * jnp.minimum(
                jnp.float32(1.0),
                _rsqrt16(jnp.maximum(ssum16, jnp.float32(1e-24))))
            vbuf[r, pl.ds(0, 16)] = v0 * scale
            vbuf[r, pl.ds(16, 16)] = v1 * scale
            return carry

        def neg_row(r, carry):
            vbuf[r, pl.ds(0, 16)] = -vbuf[r, pl.ds(0, 16)]
            vbuf[r, pl.ds(16, 16)] = -vbuf[r, pl.ds(16, 16)]
            return carry

        def clip_chunk(k, carry):
            pltpu.sync_copy(val.at[pl.ds(s * BT + k * 32, 32)], vbuf)
            lax.fori_loop(0, 32, clip_row, 0)
            pltpu.sync_copy(vbuf, stage.at[pl.ds(srow + k * 32, 32)])
            lax.fori_loop(0, 32, neg_row, 0)
            pltpu.sync_copy(
                vbuf, stage.at[pl.ds(NC * B + srow + k * 32, 32)])
            return carry

        lax.fori_loop(0, BT // 32, clip_chunk, 0)

        def accz_body(i, carry):
            pltpu.sync_copy(zrow, acc.at[pl.ds(s * ZR + i * 128, 128)])
            return carry

        lax.fori_loop(0, ZR // 128, accz_body, 0)

        plsc.subcore_barrier()

        def pass_body(p, carry):
            lo = (c * P + p) * R

            lane = lax.iota(jnp.int32, 16)

            def map_k(k, carry2):
                def map_o(o, carry3):
                    iv = idx_v[k, pl.ds(o * 16, 16)]
                    inr = (iv >= lo) & (iv < lo + R)
                    loc = iv - lo
                    g = k * 128 + o * 16
                    pos = g + lane
                    gidx[pl.ds(g, 16)] = jnp.where(inr, iv, lo + pos)
                    aidx[pl.ds(g, 16)] = jnp.where(inr, loc, pos)
                    tidx[pl.ds(g, 16)] = jnp.where(inr, loc, R + pos)
                    return carry3
                return lax.fori_loop(0, 8, map_o, carry2)

            lax.fori_loop(0, 8, map_k, 0)

            pltpu.sync_copy(stage.at[pl.ds(srow, BT)], buf)
            pltpu.async_copy(buf, acc.at[tidx], sem_a, add=True).wait()
            plsc.subcore_barrier()

            pltpu.async_copy(pristine.at[gidx], buf, sem_a).wait()
            pltpu.async_copy(acc.at[aidx], buf, sem_b, add=True).wait()
            plsc.subcore_barrier()

            pltpu.async_copy(buf, out.at[gidx], sem_b).wait()

            pltpu.sync_copy(stage.at[pl.ds(NC * B + srow, BT)], buf)
            pltpu.async_copy(buf, acc.at[tidx], sem_a, add=True).wait()
            plsc.subcore_barrier()
            return carry

        lax.fori_loop(0, P, pass_body, 0)

    return _pl_mpmd._mpmd_map(
        [(mesh, _sc_update)],
        out_types=(
            jax.ShapeDtypeStruct((M, D), jnp.float32),
            jax.ShapeDtypeStruct((2 * NC * B, D), jnp.float32),
        ),
        input_output_aliases={0: 0},
        scratch_types=[
            pltpu.VMEM_SHARED((ACC, D), jnp.float32),
            pltpu.SemaphoreType.DMA,
            pltpu.SemaphoreType.DMA,
        ],
        compiler_params=pltpu.CompilerParams(
            needs_layout_passes=False, use_tc_tiling_on_sc=False),
    )


def kernel(items_emb, idx, val):
    idx2 = idx.astype(jnp.int32).reshape(128, 128)
    out, _ = _build_sc_update()(items_emb, items_emb, idx2, val)
    return out


# speedup vs baseline: 2.4098x; 1.0957x over previous
import functools

import jax
import jax.numpy as jnp
from jax import lax
from jax.experimental import pallas as pl
from jax.experimental.pallas import tpu as pltpu
from jax.experimental.pallas import tpu_sc as plsc
from jax._src.pallas import mpmd as _pl_mpmd

M = 1_000_000
D = 32
B = 16384
LR = 0.01

NC = 2
NS = 16
BT = B // NS
ACC = 43008
W = 1024
R = 41984
P = 12
ZR = ACC // NS


def _rsqrt16(x):
    i = plsc.bitcast(x, jnp.int32)
    i = jnp.int32(0x5F3759DF) - lax.shift_right_logical(i, 1)
    y = plsc.bitcast(i, jnp.float32)
    for _ in range(3):
        y = y * (jnp.float32(1.5) - jnp.float32(0.5) * x * y * y)
    return y


@functools.cache
def _build_sc_update():
    mesh = plsc.VectorSubcoreMesh(core_axis_name="c", subcore_axis_name="s")

    def _sc_update(items, pristine, idx2, val, out, stage, acc, sem_a, sem_b):
        @pl.with_scoped(
            pltpu.VMEM((8, 128), jnp.int32),
            pltpu.VMEM((BT,), jnp.int32),
            pltpu.VMEM((BT,), jnp.int32),
            pltpu.VMEM((BT,), jnp.int32),
            pltpu.VMEM((BT, D), jnp.float32),
            pltpu.VMEM((32, D), jnp.float32),
            pltpu.VMEM((128, D), jnp.float32),
        )
        def _body(idx_v, gidx, aidx, tidx, buf, vbuf, zrow):
            _tile_body(pristine, idx2, val, out, stage, acc, sem_a, sem_b,
                       idx_v, gidx, aidx, tidx, buf, vbuf, zrow)

        _body()

    def _tile_body(pristine, idx2, val, out, stage, acc, sem_a, sem_b,
                   idx_v, gidx, aidx, tidx, buf, vbuf, zrow):
        c = lax.axis_index("c")
        s = lax.axis_index("s")

        pltpu.sync_copy(idx2.at[pl.ds(s * 8, 8)], idx_v)

        zero16 = jnp.zeros((16,), jnp.float32)

        def zrow_body(i, carry):
            zrow[i, pl.ds(0, 16)] = zero16
            zrow[i, pl.ds(16, 16)] = zero16
            return carry

        lax.fori_loop(0, 128, zrow_body, 0)

        srow = c * B + s * BT

        def clip_row(r, carry):
            v0 = vbuf[r, pl.ds(0, 16)]
            v1 = vbuf[r, pl.ds(16, 16)]
            ssum = jnp.sum(v0 * v0 + v1 * v1)
            ssum16 = jnp.broadcast_to(ssum, (16,))
            scale = jnp.float32(-LR) * jnp.minimum(
                jnp.float32(1.0),
                _rsqrt16(jnp.maximum(ssum16, jnp.float32(1e-24))))
            vbuf[r, pl.ds(0, 16)] = v0 * scale
            vbuf[r, pl.ds(16, 16)] = v1 * scale
            return carry

        def neg_row(r, carry):
            vbuf[r, pl.ds(0, 16)] = -vbuf[r, pl.ds(0, 16)]
            vbuf[r, pl.ds(16, 16)] = -vbuf[r, pl.ds(16, 16)]
            return carry

        def clip_chunk(k, carry):
            pltpu.sync_copy(val.at[pl.ds(s * BT + k * 32, 32)], vbuf)
            lax.fori_loop(0, 32, clip_row, 0)
            pltpu.sync_copy(vbuf, stage.at[pl.ds(srow + k * 32, 32)])
            lax.fori_loop(0, 32, neg_row, 0)
            pltpu.sync_copy(
                vbuf, stage.at[pl.ds(NC * B + srow + k * 32, 32)])
            return carry

        lax.fori_loop(0, BT // 32, clip_chunk, 0)

        def accz_body(i, carry):
            pltpu.sync_copy(zrow, acc.at[pl.ds(s * ZR + i * 128, 128)])
            return carry

        lax.fori_loop(0, ZR // 128, accz_body, 0)

        plsc.subcore_barrier()

        def pass_body(p, carry):
            lo = (c * P + p) * R

            lane = lax.iota(jnp.int32, 16)

            def map_k(k, carry2):
                def map_o(o, carry3):
                    iv = idx_v[k, pl.ds(o * 16, 16)]
                    inr = (iv >= lo) & (iv < lo + R)
                    loc = iv - lo
                    g = k * 128 + o * 16
                    pos = g + lane
                    gidx[pl.ds(g, 16)] = jnp.where(inr, iv, lo + pos)
                    aidx[pl.ds(g, 16)] = jnp.where(inr, loc, pos)
                    tidx[pl.ds(g, 16)] = jnp.where(inr, loc, R + pos)
                    return carry3
                return lax.fori_loop(0, 8, map_o, carry2)

            lax.fori_loop(0, 8, map_k, 0)

            pltpu.sync_copy(stage.at[pl.ds(srow, BT)], buf)
            pltpu.async_copy(buf, acc.at[tidx], sem_a, add=True).wait()
            plsc.subcore_barrier()

            pltpu.async_copy(pristine.at[gidx], buf, sem_a).wait()
            pltpu.async_copy(acc.at[aidx], buf, sem_b, add=True).wait()
            plsc.subcore_barrier()

            pltpu.async_copy(buf, out.at[gidx], sem_b).wait()

            pltpu.sync_copy(stage.at[pl.ds(NC * B + srow, BT)], buf)
            pltpu.async_copy(buf, acc.at[tidx], sem_a, add=True).wait()
            plsc.subcore_barrier()
            return carry

        lax.fori_loop(0, P, pass_body, 0)

    return _pl_mpmd._mpmd_map(
        [(mesh, _sc_update)],
        out_types=(
            jax.ShapeDtypeStruct((M, D), jnp.float32),
            jax.ShapeDtypeStruct((2 * NC * B, D), jnp.float32),
        ),
        input_output_aliases={0: 0},
        scratch_types=[
            pltpu.VMEM_SHARED((ACC, D), jnp.float32),
            pltpu.SemaphoreType.DMA,
            pltpu.SemaphoreType.DMA,
        ],
        compiler_params=pltpu.CompilerParams(
            needs_layout_passes=False, use_tc_tiling_on_sc=False),
    )


@functools.cache
def _build_tc_copy():
    BLK = 8000

    def body(in_ref, out_ref):
        out_ref[...] = in_ref[...]

    return pl.pallas_call(
        body,
        grid=(M // BLK,),
        in_specs=[pl.BlockSpec((BLK, D), lambda i: (i, 0))],
        out_specs=pl.BlockSpec((BLK, D), lambda i: (i, 0)),
        out_shape=jax.ShapeDtypeStruct((M, D), jnp.float32),
    )


def kernel(items_emb, idx, val):
    idx2 = idx.astype(jnp.int32).reshape(128, 128)
    tbl = _build_tc_copy()(items_emb)
    out, _ = _build_sc_update()(tbl, items_emb, idx2, val)
    return out


# speedup vs baseline: 2.8748x; 1.1930x over previous
import functools

import jax
import jax.numpy as jnp
from jax import lax
from jax.experimental import pallas as pl
from jax.experimental.pallas import tpu as pltpu
from jax.experimental.pallas import tpu_sc as plsc
from jax._src.pallas import mpmd as _pl_mpmd

M = 1_000_000
D = 32
B = 16384
LR = 0.01

NC = 2
NS = 16
BT = B // NS
ACC = 43008
W = 1024
R = 41984
P = 12
ZR = ACC // NS


def _rsqrt16(x):
    i = plsc.bitcast(x, jnp.int32)
    i = jnp.int32(0x5F3759DF) - lax.shift_right_logical(i, 1)
    y = plsc.bitcast(i, jnp.float32)
    for _ in range(3):
        y = y * (jnp.float32(1.5) - jnp.float32(0.5) * x * y * y)
    return y


@functools.cache
def _build_sc_update():
    mesh = plsc.VectorSubcoreMesh(core_axis_name="c", subcore_axis_name="s")

    def _sc_update(items, pristine, idx2, val, out, stage, acc, sem_a, sem_b):
        @pl.with_scoped(
            pltpu.VMEM((8, 128), jnp.int32),
            pltpu.VMEM((BT,), jnp.int32),
            pltpu.VMEM((BT,), jnp.int32),
            pltpu.VMEM((BT,), jnp.int32),
            pltpu.VMEM((BT, D), jnp.float32),
            pltpu.VMEM((32, D), jnp.float32),
            pltpu.VMEM((128, D), jnp.float32),
        )
        def _body(idx_v, gidx, aidx, tidx, buf, vbuf, zrow):
            _tile_body(pristine, idx2, val, out, stage, acc, sem_a, sem_b,
                       idx_v, gidx, aidx, tidx, buf, vbuf, zrow)

        _body()

    def _tile_body(pristine, idx2, val, out, stage, acc, sem_a, sem_b,
                   idx_v, gidx, aidx, tidx, buf, vbuf, zrow):
        c = lax.axis_index("c")
        s = lax.axis_index("s")

        pltpu.sync_copy(idx2.at[pl.ds(s * 8, 8)], idx_v)

        zero16 = jnp.zeros((16,), jnp.float32)

        def zrow_body(i, carry):
            zrow[i, pl.ds(0, 16)] = zero16
            zrow[i, pl.ds(16, 16)] = zero16
            return carry

        lax.fori_loop(0, 128, zrow_body, 0)

        srow = c * B + s * BT

        def clip_row(r, carry):
            v0 = vbuf[r, pl.ds(0, 16)]
            v1 = vbuf[r, pl.ds(16, 16)]
            ssum = jnp.sum(v0 * v0 + v1 * v1)
            ssum16 = jnp.broadcast_to(ssum, (16,))
            scale = jnp.float32(-LR) * jnp.minimum(
                jnp.float32(1.0),
                _rsqrt16(jnp.maximum(ssum16, jnp.float32(1e-24))))
            vbuf[r, pl.ds(0, 16)] = v0 * scale
            vbuf[r, pl.ds(16, 16)] = v1 * scale
            return carry

        def neg_row(r, carry):
            vbuf[r, pl.ds(0, 16)] = -vbuf[r, pl.ds(0, 16)]
            vbuf[r, pl.ds(16, 16)] = -vbuf[r, pl.ds(16, 16)]
            return carry

        def clip_chunk(k, carry):
            pltpu.sync_copy(val.at[pl.ds(s * BT + k * 32, 32)], vbuf)
            lax.fori_loop(0, 32, clip_row, 0)
            pltpu.sync_copy(vbuf, stage.at[pl.ds(srow + k * 32, 32)])
            lax.fori_loop(0, 32, neg_row, 0)
            pltpu.sync_copy(
                vbuf, stage.at[pl.ds(NC * B + srow + k * 32, 32)])
            return carry

        lax.fori_loop(0, BT // 32, clip_chunk, 0)

        def accz_body(i, carry):
            pltpu.sync_copy(zrow, acc.at[pl.ds(s * ZR + i * 128, 128)])
            return carry

        lax.fori_loop(0, ZR // 128, accz_body, 0)

        plsc.subcore_barrier()

        def pass_body(p, carry):
            lo = (c * P + p) * R

            lane = lax.iota(jnp.int32, 16)

            def map_k(k, carry2):
                def map_o(o, carry3):
                    iv = idx_v[k, pl.ds(o * 16, 16)]
                    inr = (iv >= lo) & (iv < lo + R)
                    loc = iv - lo
                    g = k * 128 + o * 16
                    pos = g + lane
                    gidx[pl.ds(g, 16)] = jnp.where(inr, iv, lo + pos)
                    aidx[pl.ds(g, 16)] = jnp.where(inr, loc, pos)
                    tidx[pl.ds(g, 16)] = jnp.where(inr, loc, R + pos)
                    return carry3
                return lax.fori_loop(0, 8, map_o, carry2)

            lax.fori_loop(0, 8, map_k, 0)

            pltpu.sync_copy(stage.at[pl.ds(srow, BT)], buf)
            pltpu.async_copy(buf, acc.at[tidx], sem_a, add=True).wait()
            plsc.subcore_barrier()

            pltpu.async_copy(pristine.at[gidx], buf, sem_a).wait()
            pltpu.async_copy(acc.at[aidx], buf, sem_b, add=True).wait()
            plsc.subcore_barrier()

            pltpu.async_copy(buf, out.at[gidx], sem_b).wait()

            pltpu.sync_copy(stage.at[pl.ds(NC * B + srow, BT)], buf)
            pltpu.async_copy(buf, acc.at[tidx], sem_a, add=True).wait()
            plsc.subcore_barrier()
            return carry

        lax.fori_loop(0, P, pass_body, 0)

    return _pl_mpmd._mpmd_map(
        [(mesh, _sc_update)],
        out_types=(
            jax.ShapeDtypeStruct((M, D), jnp.float32),
            jax.ShapeDtypeStruct((2 * NC * B, D), jnp.float32),
        ),
        input_output_aliases={0: 0},
        scratch_types=[
            pltpu.VMEM_SHARED((ACC, D), jnp.float32),
            pltpu.SemaphoreType.DMA,
            pltpu.SemaphoreType.DMA,
        ],
        compiler_params=pltpu.CompilerParams(
            needs_layout_passes=False, use_tc_tiling_on_sc=False),
    )


@functools.cache
def _build_tc_copy():
    ROWS = M * D // 128
    BLK = 2000

    def body(in_ref, out_ref):
        out_ref[...] = in_ref[...]

    return pl.pallas_call(
        body,
        grid=(ROWS // BLK,),
        in_specs=[pl.BlockSpec((BLK, 128), lambda i: (i, 0))],
        out_specs=pl.BlockSpec((BLK, 128), lambda i: (i, 0)),
        out_shape=jax.ShapeDtypeStruct((ROWS, 128), jnp.float32),
    )


def kernel(items_emb, idx, val):
    idx2 = idx.astype(jnp.int32).reshape(128, 128)
    tbl = _build_tc_copy()(items_emb.reshape(M * D // 128, 128))
    tbl = tbl.reshape(M, D)
    out, _ = _build_sc_update()(tbl, items_emb, idx2, val)
    return out
